# Initial kernel scaffold; baseline (speedup 1.0000x reference)
import jax
import jax.numpy as jnp
from jax.experimental import pallas as pl


def kernel(Imagetype, neighbor_list, ImagedR, n_em, W1, b1, W2, b2, W3, b3, W4, b4):
    raise NotImplementedError("write your pallas kernel here")


# speedup vs baseline: 10.5611x; 10.5611x over previous
import functools

import jax
import jax.numpy as jnp
import numpy as np
from jax import lax
from jax.experimental import pallas as pl
from jax.experimental.pallas import tpu as pltpu
from jax.experimental.pallas import tpu_sc as plsc

_B, _N, _M, _NB = 2, 256, 64, 25
_E = _M * _N
_RMAX, _RMIN = 6.0, 0.5
_PI = 3.141592653589
_ROWS = 272
_ACC = 3 * _ROWS
_NT = 16
_EPT = _E // _NT
_RPT = _EPT // 128


def _sc_gather_body(aa_hbm, nl_hbm, t_hbm, aa_v, nl_v, t_v):
    b = lax.axis_index("c")
    s = lax.axis_index("s")
    pltpu.sync_copy(aa_hbm.at[b], aa_v)
    pltpu.sync_copy(nl_hbm.at[b, pl.ds(s * _RPT, _RPT)], nl_v)
    for r in range(_RPT):
        for j in range(8):
            idx = nl_v[r, pl.ds(j * 16, 16)]
            t_v[r, pl.ds(j * 16, 16)] = plsc.load_gather(aa_v, [idx])
    pltpu.sync_copy(t_v, t_hbm.at[b, pl.ds(s * _RPT, _RPT)])


_sc_gather = pl.kernel(
    _sc_gather_body,
    out_type=jax.ShapeDtypeStruct((_B, _E // 128, 128), jnp.int32),
    mesh=plsc.VectorSubcoreMesh(core_axis_name="c", subcore_axis_name="s"),
    compiler_params=pltpu.CompilerParams(needs_layout_passes=False),
    scratch_types=[
        pltpu.VMEM((264,), jnp.int32),
        pltpu.VMEM((_RPT, 128), jnp.int32),
        pltpu.VMEM((_RPT, 128), jnp.int32),
    ],
)


def _sc_scatter_body(nl_hbm, v_hbm, idx2_hbm, out_hbm,
                     nl_v, i0, i1, i2, j0, j1, j2, val_v, val2_v, zbuf, acc):
    b = lax.axis_index("c")
    s = lax.axis_index("s")

    @pl.when(s == 0)
    def _():
        for i in range(_ACC // 16):
            zbuf[pl.ds(i * 16, 16)] = jnp.zeros((16,), jnp.float32)
        pltpu.sync_copy(zbuf, acc)

    idx = [i0, i1, i2]
    idx2 = [j0, j1, j2]
    pltpu.sync_copy(nl_hbm.at[b, s], nl_v)
    for c in range(3):
        pltpu.sync_copy(idx2_hbm.at[c], idx2[c])
        for i in range(_EPT // 16):
            sl = pl.ds(i * 16, 16)
            idx[c][sl] = nl_v[sl] + (c * _ROWS)
    plsc.subcore_barrier()
    for c in range(3):
        pltpu.sync_copy(v_hbm.at[b, c, s], val_v)
        for i in range(_EPT // 16):
            sl = pl.ds(i * 16, 16)
            val2_v[sl] = -val_v[sl]
        pltpu.sync_copy(val_v, acc.at[idx[c]], add=True)
        pltpu.sync_copy(val2_v, acc.at[idx2[c]], add=True)
    plsc.subcore_barrier()

    @pl.when(s == 0)
    def _():
        pltpu.sync_copy(acc, out_hbm.at[b])


_sc_scatter = pl.kernel(
    _sc_scatter_body,
    out_type=jax.ShapeDtypeStruct((_B, _ACC), jnp.float32),
    mesh=plsc.VectorSubcoreMesh(core_axis_name="c", subcore_axis_name="s"),
    compiler_params=pltpu.CompilerParams(use_tc_tiling_on_sc=False),
    scratch_types=[
        pltpu.VMEM((_EPT,), jnp.int32),
        pltpu.VMEM((_EPT,), jnp.int32),
        pltpu.VMEM((_EPT,), jnp.int32),
        pltpu.VMEM((_EPT,), jnp.int32),
        pltpu.VMEM((_EPT,), jnp.int32),
        pltpu.VMEM((_EPT,), jnp.int32),
        pltpu.VMEM((_EPT,), jnp.int32),
        pltpu.VMEM((_EPT,), jnp.float32),
        pltpu.VMEM((_EPT,), jnp.float32),
        pltpu.VMEM((_ACC,), jnp.float32),
        pltpu.VMEM_SHARED((_ACC,), jnp.float32),
    ],
)


def _tc_body(rT_ref, t_ref, ti_ref, em_ref,
             W1s_ref, W1sT_ref, W2_ref, W2T_ref, W3_ref, W3T_ref,
             W4_ref, W4T_ref, b1_ref, b2_ref, b3_ref, b4_ref,
             Ei_ref, Etot_ref, v_ref):
    f32 = jnp.float32
    r = rT_ref[0, 0]
    xx = rT_ref[0, 1]
    yy = rT_ref[0, 2]
    zz = rT_ref[0, 3]
    t = t_ref[0]
    ti = ti_ref[0]

    cond = (r < _RMAX) & (r > 0.0)
    nz = r != 0.0
    safe = jnp.where(nz, r, 1.0)
    inv = 1.0 / safe
    ux = jnp.where(nz, xx * inv, 0.0)
    uy = jnp.where(nz, yy * inv, 0.0)
    uz = jnp.where(nz, zz * inv, 0.0)
    env = 0.5 * jnp.cos(_PI * r) + 0.5
    denv = (-0.5 * _PI) * jnp.sin(_PI * r)

    ks = lax.broadcasted_iota(jnp.int32, (1, _NB, 1), 1).astype(f32)
    rs3 = _RMIN + ks * ((_RMAX - _RMIN) / (_NB - 1))
    d3 = r[:, None, :] - rs3
    e3 = jnp.exp(-d3 * d3)
    c3 = cond[:, None, :]
    env3 = env[:, None, :]
    fc3 = jnp.where(c3, e3 * env3, 0.0)
    dfc3 = jnp.where(c3, e3 * (-2.0 * d3 * env3 + denv[:, None, :]), 0.0)

    oh = [(t == c).astype(f32) for c in range(5)]
    ohn = [(ti == c).astype(f32) for c in range(5)]
    F = [jnp.sum(fc3 * oh[c][:, None, :], axis=0) for c in range(5)]
    em = [[em_ref[c, q] for q in range(5)] for c in range(5)]
    S = [sum(F[c] * em[c][q] for c in range(5)) for q in range(5)]
    ii = [sum(ohn[c] * em[c][p] for c in range(5)) for p in range(5)]

    dot = functools.partial(jnp.dot, preferred_element_type=f32)
    x1 = b1_ref[...]
    for j in range(25):
        x1 = x1 + ii[j // 5] * dot(W1s_ref[j], S[j % 5])
    x2 = jnp.tanh(x1)
    x3 = dot(W2_ref[...], x2) + b2_ref[...] + x1
    x4 = jnp.tanh(x3)
    x5 = dot(W3_ref[...], x4) + b3_ref[...] + x3
    xa = jnp.tanh(x5)
    Ei = dot(W4_ref[...], xa) + b4_ref[...]
    Ei_ref[0] = Ei
    Etot_ref[0] = jnp.broadcast_to(jnp.sum(Ei), (1, _N))

    gx5 = W4T_ref[...] * (1.0 - xa * xa)
    gx4 = dot(W3T_ref[...], gx5)
    gx3 = gx4 * (1.0 - x4 * x4) + gx5
    gx2 = dot(W2T_ref[...], gx3)
    gx1 = gx2 * (1.0 - x2 * x2) + gx3
    gG = [dot(W1sT_ref[j], gx1) for j in range(25)]
    H = [sum(gG[5 * p + q] * ii[p] for p in range(5)) for q in range(5)]
    P = [sum(H[q] * em[c][q] for q in range(5)) for c in range(5)]
    D = [jnp.sum(dfc3 * P[c][None, :, :], axis=1) for c in range(5)]
    w = sum(oh[c] * D[c] for c in range(5))
    v_ref[0, 0] = w * ux
    v_ref[0, 1] = w * uy
    v_ref[0, 2] = w * uz


def _tc_dense(rT, t, ti3, em, W1s, W1sT, W2, W2T, W3, W3T, W4, W4T,
              b1c, b2c, b3c, b4c, interpret=False):
    full = lambda shape: pl.BlockSpec(shape, lambda b: (0,) * len(shape))
    return pl.pallas_call(
        _tc_body,
        grid=(_B,),
        in_specs=[
            pl.BlockSpec((1, 4, _M, _N), lambda b: (b, 0, 0, 0)),
            pl.BlockSpec((1, _M, _N), lambda b: (b, 0, 0)),
            pl.BlockSpec((1, 1, _N), lambda b: (b, 0, 0)),
            pl.BlockSpec(memory_space=pltpu.SMEM),
            full((25, 100, 25)), full((25, 25, 100)),
            full((100, 100)), full((100, 100)),
            full((100, 100)), full((100, 100)),
            full((1, 100)), full((100, 1)),
            full((100, 1)), full((100, 1)), full((100, 1)), full((1, 1)),
        ],
        out_specs=[
            pl.BlockSpec((1, 1, _N), lambda b: (b, 0, 0)),
            pl.BlockSpec((1, 1, _N), lambda b: (b, 0, 0)),
            pl.BlockSpec((1, 3, _M, _N), lambda b: (b, 0, 0, 0)),
        ],
        out_shape=[
            jax.ShapeDtypeStruct((_B, 1, _N), jnp.float32),
            jax.ShapeDtypeStruct((_B, 1, _N), jnp.float32),
            jax.ShapeDtypeStruct((_B, 3, _M, _N), jnp.float32),
        ],
        interpret=interpret,
    )(rT, t, ti3, em, W1s, W1sT, W2, W2T, W3, W3T, W4, W4T,
      b1c, b2c, b3c, b4c)


_IDX2 = np.asarray(
    np.arange(3)[:, None] * _ROWS + (np.arange(_EPT)[None] % _N) + 1,
    dtype=np.int32)


def kernel(Imagetype, neighbor_list, ImagedR, n_em, W1, b1, W2, b2, W3, b3, W4, b4):
    i32 = jnp.int32
    ti = Imagetype.astype(i32)
    nlE = neighbor_list.astype(i32).transpose(0, 2, 1).reshape(_B, _E // 128, 128)
    aa = jnp.concatenate(
        [jnp.zeros((_B, 1), i32), ti, jnp.zeros((_B, 264 - _N - 1), i32)], axis=1)

    t_flat = _sc_gather(aa, nlE)
    t = t_flat.reshape(_B, _M, _N)

    rT = ImagedR.transpose(0, 3, 2, 1)
    ti3 = ti.reshape(_B, 1, _N)
    em = n_em[:5, :5]
    W1s = W1.reshape(100, _NB, 25).transpose(2, 0, 1)
    W1sT = W1s.transpose(0, 2, 1)
    Ei3, Etot3, v4 = _tc_dense(
        rT, t, ti3, em, W1s, W1sT, W2, W2.T, W3, W3.T, W4, W4.T,
        b1.reshape(100, 1), b2.reshape(100, 1), b3.reshape(100, 1),
        b4.reshape(1, 1))

    vE = v4.reshape(_B, 3, _NT, _EPT)
    nlS = nlE.reshape(_B, _NT, _EPT)
    out = _sc_scatter(nlS, vE, jnp.asarray(_IDX2))
    Force = out.reshape(_B, 3, _ROWS)[:, :, 1:_N + 1].transpose(0, 2, 1)
    return Etot3[:, 0, :1], Ei3.transpose(0, 2, 1), Force


# speedup vs baseline: 11.8874x; 1.1256x over previous
import functools

import jax
import jax.numpy as jnp
import numpy as np
from jax import lax
from jax.experimental import pallas as pl
from jax.experimental.pallas import tpu as pltpu
from jax.experimental.pallas import tpu_sc as plsc

_B, _N, _M, _NB = 2, 256, 64, 25
_E = _M * _N
_RMAX, _RMIN = 6.0, 0.5
_PI = 3.141592653589
_ROWS = 336
_SROW = 1024
_NT = 16
_EPT = _E // _NT
_RPT = _EPT // 128
_KP = 32


def _sc_gather_body(aa_hbm, nl_hbm, t_hbm, aa_v, nl_v, t_v):
    b = lax.axis_index("c")
    s = lax.axis_index("s")
    pltpu.sync_copy(aa_hbm.at[b], aa_v)
    pltpu.sync_copy(nl_hbm.at[b, pl.ds(s * _RPT, _RPT)], nl_v)
    for r in range(_RPT):
        for j in range(8):
            idx = nl_v[r, pl.ds(j * 16, 16)]
            t_v[r, pl.ds(j * 16, 16)] = plsc.load_gather(aa_v, [idx])
    pltpu.sync_copy(t_v, t_hbm.at[b, pl.ds(s * _RPT, _RPT)])


_sc_gather = pl.kernel(
    _sc_gather_body,
    out_type=jax.ShapeDtypeStruct((_B, _E // 128, 128), jnp.int32),
    mesh=plsc.VectorSubcoreMesh(core_axis_name="c", subcore_axis_name="s"),
    compiler_params=pltpu.CompilerParams(needs_layout_passes=False),
    scratch_types=[
        pltpu.VMEM((264,), jnp.int32),
        pltpu.VMEM((_RPT, 128), jnp.int32),
        pltpu.VMEM((_RPT, 128), jnp.int32),
    ],
)


def _sc_scatter_body(nl_hbm, v_hbm, idx2_hbm, out_hbm,
                     nl_v, i0, i1, i2, j0, j1, j2, v0, v1, v2,
                     w0, w1, w2, zbuf, rbuf, obuf, slab, sem_in, sem_rd):
    b = lax.axis_index("c")
    s = lax.axis_index("s")
    base = s * _SROW

    nl_d = pltpu.async_copy(nl_hbm.at[b, s], nl_v, sem_in)
    idx2 = [j0, j1, j2]
    val = [v0, v1, v2]
    neg = [w0, w1, w2]
    idx2_d = [pltpu.async_copy(idx2_hbm.at[c], idx2[c], sem_in) for c in range(3)]
    val_d = [pltpu.async_copy(v_hbm.at[b, c, s], val[c], sem_in) for c in range(3)]

    for i in range(_SROW // 16):
        zbuf[pl.ds(i * 16, 16)] = jnp.zeros((16,), jnp.float32)
    pltpu.sync_copy(zbuf, slab.at[pl.ds(s * _SROW, _SROW)])

    idx = [i0, i1, i2]
    nl_d.wait()
    for c in range(3):
        idx2_d[c].wait()
    for i in range(_EPT // 16):
        sl = pl.ds(i * 16, 16)
        nl16 = nl_v[sl]
        for c in range(3):
            idx[c][sl] = nl16 + (base + c * _ROWS)
            idx2[c][sl] = idx2[c][sl] + base
    for c in range(3):
        val_d[c].wait()
        for i in range(_EPT // 16):
            sl = pl.ds(i * 16, 16)
            neg[c][sl] = -val[c][sl]
        pltpu.sync_copy(val[c], slab.at[idx[c]], add=True)
        pltpu.sync_copy(neg[c], slab.at[idx2[c]], add=True)
    plsc.subcore_barrier()
    rd = [pltpu.async_copy(slab.at[pl.ds(r * _SROW + 64 * s, 64)],
                           rbuf.at[pl.ds(64 * r, 64)], sem_rd)
          for r in range(_NT)]
    for d in rd:
        d.wait()
    for j in range(4):
        sl = pl.ds(16 * j, 16)
        acc16 = rbuf[sl]
        for r in range(1, _NT):
            acc16 = acc16 + rbuf[pl.ds(64 * r + 16 * j, 16)]
        obuf[sl] = acc16
    pltpu.sync_copy(obuf, out_hbm.at[b, pl.ds(64 * s, 64)])


_sc_scatter = pl.kernel(
    _sc_scatter_body,
    out_type=jax.ShapeDtypeStruct((_B, _SROW), jnp.float32),
    mesh=plsc.VectorSubcoreMesh(core_axis_name="c", subcore_axis_name="s"),
    compiler_params=pltpu.CompilerParams(use_tc_tiling_on_sc=False),
    scratch_types=(
        [pltpu.VMEM((_EPT,), jnp.int32) for _ in range(7)]
        + [pltpu.VMEM((_EPT,), jnp.float32) for _ in range(6)]
        + [pltpu.VMEM((_SROW,), jnp.float32),
           pltpu.VMEM((64 * _NT,), jnp.float32),
           pltpu.VMEM((64,), jnp.float32),
           pltpu.VMEM_SHARED((_NT * _SROW,), jnp.float32),
           pltpu.SemaphoreType.DMA,
           pltpu.SemaphoreType.DMA]
    ),
)


def _bf(x):
    return x.astype(jnp.bfloat16)


def _dot(a, b):
    return jnp.dot(_bf(a), _bf(b), preferred_element_type=jnp.float32)


def _dotT(a, b):
    return lax.dot_general(_bf(a), _bf(b), (((0,), (0,)), ((), ())),
                           preferred_element_type=jnp.float32)


def _tc_body(rT_ref, t_ref, ti_ref, em_ref,
             W1_ref, W1T_ref, W2_ref, W3_ref,
             W4_ref, b1_ref, b2_ref, b3_ref, b4_ref,
             Ei_ref, Etot_ref, v_ref):
    f32 = jnp.float32
    r = rT_ref[0, 0]
    xx = rT_ref[0, 1]
    yy = rT_ref[0, 2]
    zz = rT_ref[0, 3]
    t = t_ref[0]
    ti = ti_ref[0]

    cond = (r < _RMAX) & (r > 0.0)
    nz = r != 0.0
    safe = jnp.where(nz, r, 1.0)
    inv = 1.0 / safe
    ux = jnp.where(nz, xx * inv, 0.0)
    uy = jnp.where(nz, yy * inv, 0.0)
    uz = jnp.where(nz, zz * inv, 0.0)
    env = 0.5 * jnp.cos(_PI * r) + 0.5
    denv = (-0.5 * _PI) * jnp.sin(_PI * r)

    ki = lax.broadcasted_iota(jnp.int32, (1, _KP, 1), 1)
    ks = ki.astype(f32)
    rs3 = _RMIN + ks * ((_RMAX - _RMIN) / (_NB - 1))
    d3 = r[:, None, :] - rs3
    e3 = jnp.exp(-d3 * d3)
    c3 = cond[:, None, :] & (ki < _NB)
    env3 = env[:, None, :]
    fc3 = jnp.where(c3, e3 * env3, 0.0)
    dfc3 = jnp.where(c3, e3 * (-2.0 * d3 * env3 + denv[:, None, :]), 0.0)

    oh = [(t == c).astype(f32) for c in range(5)]
    ohn = [(ti == c).astype(f32) for c in range(5)]
    F = [jnp.sum(fc3 * oh[c][:, None, :], axis=0) for c in range(5)]
    em = [[em_ref[c, q] for q in range(5)] for c in range(5)]
    S = [sum(F[c] * em[c][q] for c in range(5)) for q in range(5)]
    ii = [sum(ohn[c] * em[c][p] for c in range(5)) for p in range(5)]

    G = jnp.concatenate([ii[j // 5] * S[j % 5] for j in range(25)], axis=0)
    x1 = _dot(W1_ref[...], G) + b1_ref[...]
    x2 = jnp.tanh(x1)
    x3 = _dot(W2_ref[...], x2) + b2_ref[...] + x1
    x4 = jnp.tanh(x3)
    x5 = _dot(W3_ref[...], x4) + b3_ref[...] + x3
    xa = jnp.tanh(x5)
    Ei = _dot(W4_ref[...], xa) + b4_ref[...]
    Ei_ref[0] = Ei
    Etot_ref[0] = jnp.broadcast_to(jnp.sum(Ei), (1, _N))

    gxa = lax.dot_general(W4_ref[...], jnp.ones((1, _N), f32),
                          (((0,), (0,)), ((), ())),
                          preferred_element_type=f32)
    gx5 = gxa * (1.0 - xa * xa)
    gx4 = _dotT(W3_ref[...], gx5)
    gx3 = gx4 * (1.0 - x4 * x4) + gx5
    gx2 = _dotT(W2_ref[...], gx3)
    gx1 = gx2 * (1.0 - x2 * x2) + gx3
    gG = _dot(W1T_ref[...], gx1)
    H = [sum(gG[_KP * (5 * p + q):_KP * (5 * p + q) + _KP] * ii[p]
             for p in range(5)) for q in range(5)]
    P = [sum(H[q] * em[c][q] for q in range(5)) for c in range(5)]
    D = [jnp.sum(dfc3 * P[c][None, :, :], axis=1) for c in range(5)]
    w = sum(oh[c] * D[c] for c in range(5))
    v_ref[0, 0] = w * ux
    v_ref[0, 1] = w * uy
    v_ref[0, 2] = w * uz


def _tc_dense(rT, t, ti3, em, W1p, W1pT, W2, W3, W4,
              b1c, b2c, b3c, b4c, interpret=False):
    full = lambda shape: pl.BlockSpec(shape, lambda b: (0,) * len(shape))
    return pl.pallas_call(
        _tc_body,
        grid=(_B,),
        in_specs=[
            pl.BlockSpec((1, 4, _M, _N), lambda b: (b, 0, 0, 0)),
            pl.BlockSpec((1, _M, _N), lambda b: (b, 0, 0)),
            pl.BlockSpec((1, 1, _N), lambda b: (b, 0, 0)),
            pl.BlockSpec(memory_space=pltpu.SMEM),
            full((100, 25 * _KP)), full((25 * _KP, 100)),
            full((100, 100)),
            full((100, 100)),
            full((1, 100)),
            full((100, 1)), full((100, 1)), full((100, 1)), full((1, 1)),
        ],
        compiler_params=pltpu.CompilerParams(
            dimension_semantics=("parallel",)),
        out_specs=[
            pl.BlockSpec((1, 1, _N), lambda b: (b, 0, 0)),
            pl.BlockSpec((1, 1, _N), lambda b: (b, 0, 0)),
            pl.BlockSpec((1, 3, _M, _N), lambda b: (b, 0, 0, 0)),
        ],
        out_shape=[
            jax.ShapeDtypeStruct((_B, 1, _N), jnp.float32),
            jax.ShapeDtypeStruct((_B, 1, _N), jnp.float32),
            jax.ShapeDtypeStruct((_B, 3, _M, _N), jnp.float32),
        ],
        interpret=interpret,
    )(rT, t, ti3, em, W1p, W1pT, W2, W3, W4, b1c, b2c, b3c, b4c)


_IDX2 = np.asarray(
    np.arange(3)[:, None] * _ROWS + (np.arange(_EPT)[None] % _N) + 1,
    dtype=np.int32)


def kernel(Imagetype, neighbor_list, ImagedR, n_em, W1, b1, W2, b2, W3, b3, W4, b4):
    i32 = jnp.int32
    ti = Imagetype.astype(i32)
    nlE = neighbor_list.astype(i32).transpose(0, 2, 1).reshape(_B, _E // 128, 128)
    aa = jnp.concatenate(
        [jnp.zeros((_B, 1), i32), ti, jnp.zeros((_B, 264 - _N - 1), i32)], axis=1)

    t_flat = _sc_gather(aa, nlE)
    t = t_flat.reshape(_B, _M, _N)

    rT = ImagedR.transpose(0, 3, 2, 1)
    ti3 = ti.reshape(_B, 1, _N)
    em = n_em[:5, :5]
    W1p = jnp.pad(W1.reshape(100, _NB, 25).transpose(0, 2, 1),
                  ((0, 0), (0, 0), (0, _KP - _NB))).reshape(100, 25 * _KP)
    Ei3, Etot3, v4 = _tc_dense(
        rT, t, ti3, em, W1p, W1p.T, W2, W3, W4,
        b1.reshape(100, 1), b2.reshape(100, 1), b3.reshape(100, 1),
        b4.reshape(1, 1))

    vE = v4.reshape(_B, 3, _NT, _EPT)
    nlS = nlE.reshape(_B, _NT, _EPT)
    out = _sc_scatter(nlS, vE, jnp.asarray(_IDX2))
    Force = jnp.stack([out[:, c * _ROWS + 1:c * _ROWS + 1 + _N]
                       for c in range(3)], axis=2)
    return Etot3[:, 0, :1], Ei3.transpose(0, 2, 1), Force
